# baseline (speedup 1.0000x reference)
import jax
import jax.numpy as jnp
from jax.experimental import pallas as pl

N = 320000
F = 128
GRID = 25
ROWS = N // GRID
G = ROWS // F


def _dgnn_body(x1_ref, x2_ref, out_ref):
    i = pl.program_id(0)
    prod = x1_ref[...] * x2_ref[...]
    out_ref[i, :, :] = jnp.sum(prod.reshape(G, F, F), axis=2)

    @pl.when(i == GRID - 1)
    def _():
        dots = out_ref[...]
        m = jnp.max(dots)
        e = jnp.exp(dots - m)
        out_ref[...] = e / jnp.sum(e)


def kernel(node1, node2):
    res = pl.pallas_call(
        _dgnn_body,
        grid=(GRID,),
        in_specs=[
            pl.BlockSpec((ROWS, None, F), lambda i: (i, 0, 0)),
            pl.BlockSpec((ROWS, None, F), lambda i: (i, 0, 0)),
        ],
        out_specs=pl.BlockSpec((GRID, G, F), lambda i: (0, 0, 0)),
        out_shape=jax.ShapeDtypeStruct((GRID, G, F), jnp.float32),
    )(node1, node2)
    return res.reshape(N, 1)


# speedup vs baseline: 1.3089x; 1.3089x over previous
import jax
import jax.numpy as jnp
from jax.experimental import pallas as pl

N = 320000
F = 128
GRID = 25
ROWS = N // GRID
G = ROWS // F


def _dgnn_body(x1_ref, x2_ref, out_ref):
    i = pl.program_id(0)
    prod = x1_ref[...] * x2_ref[...]
    out_ref[i, :, :] = jnp.sum(prod.reshape(G, F, F), axis=2)

    @pl.when(i == GRID - 1)
    def _():
        dots = out_ref[...]
        m = jnp.max(dots)
        e = jnp.exp(dots - m)
        out_ref[...] = e / jnp.sum(e)


def kernel(node1, node2):
    x1 = node1.reshape(N, F)
    x2 = node2.reshape(N, F)
    res = pl.pallas_call(
        _dgnn_body,
        grid=(GRID,),
        in_specs=[
            pl.BlockSpec((ROWS, F), lambda i: (i, 0)),
            pl.BlockSpec((ROWS, F), lambda i: (i, 0)),
        ],
        out_specs=pl.BlockSpec((GRID, G, F), lambda i: (0, 0, 0)),
        out_shape=jax.ShapeDtypeStruct((GRID, G, F), jnp.float32),
    )(x1, x2)
    return res.reshape(N, 1)
